# baseline (speedup 1.0000x reference)
import jax
import jax.numpy as jnp
from jax import lax
from jax.experimental import pallas as pl
from jax.experimental.pallas import tpu as pltpu, tpu_sc as plsc

NUM_NODES = 50000
D_FEAT = 256
NUM_GRAPHS = 128

NUM_CORES = 2
NUM_SUBCORES = 16
COLS_PER_CORE = D_FEAT // NUM_CORES

CHUNK = 80
NUM_CHUNKS = NUM_NODES // CHUNK
SUPER = 4
SROWS = SUPER * CHUNK
NSUPER = 4
SC_CHUNKS_PER_SUB = NSUPER * SUPER
SC_ROWS = NUM_SUBCORES * SC_CHUNKS_PER_SUB * CHUNK
SC_CHUNK0 = NUM_CHUNKS - NUM_SUBCORES * SC_CHUNKS_PER_SUB

TC_ROWS = NUM_NODES - SC_ROWS
TC_NB = 10
TC_BS = TC_ROWS // TC_NB


def _sc_pool_kernel(x_hbm, batch_hbm, out_hbm,
                    idx3d_v, rows_v, obuf_v, acc_sh,
                    gsem0, gsem1, ssem0, ssem1, isem):
    c = lax.axis_index("c")
    s = lax.axis_index("s")
    col0 = c * COLS_PER_CORE
    gsems = (gsem0, gsem1)
    ssems = (ssem0, ssem1)

    start = SC_CHUNK0 + s * SC_CHUNKS_PER_SUB

    def super_src(g):
        return x_hbm.at[pl.ds((start + g * SUPER) * CHUNK, SROWS),
                        pl.ds(col0, COLS_PER_CORE)]

    pltpu.async_copy(super_src(0), rows_v.at[0], gsems[0])

    for j in range(SC_CHUNKS_PER_SUB):
        pltpu.async_copy(
            batch_hbm.at[pl.ds((start + j) * CHUNK, CHUNK)],
            idx3d_v.at[j, 0], isem)

    zeros16 = jnp.zeros((16,), jnp.float32)
    for i in range(8):
        for j in range(COLS_PER_CORE // 16):
            obuf_v[i, pl.ds(j * 16, 16)] = zeros16
    pltpu.sync_copy(obuf_v, acc_sh.at[pl.ds(s * 8, 8), :])
    plsc.subcore_barrier()

    for j in range(SC_CHUNKS_PER_SUB):
        pltpu.make_async_copy(
            batch_hbm.at[pl.ds((start + j) * CHUNK, CHUNK)],
            idx3d_v.at[j, 0], isem).wait()

    def scatter_slices(g, b):
        for k in range(SUPER):
            yield (rows_v.at[b, pl.ds(k * CHUNK, CHUNK)],
                   acc_sh.at[idx3d_v.at[g * SUPER + k, 0]])

    def body(i, carry):
        for b in range(2):
            g = 2 * i + b
            nb = (b + 1) % 2

            @pl.when(g < NSUPER)
            def _():
                pltpu.make_async_copy(super_src(0), rows_v.at[b],
                                      gsems[b]).wait()
                for src, dst in scatter_slices(g, b):
                    pltpu.async_copy(src, dst, ssems[b], add=True)

            @pl.when((g >= 1) & (g - 1 < NSUPER))
            def _():
                for src, dst in scatter_slices(g - 1, nb):
                    pltpu.make_async_copy(src, dst, ssems[nb]).wait()

            @pl.when(g + 1 < NSUPER)
            def _():
                pltpu.async_copy(super_src(g + 1), rows_v.at[nb], gsems[nb])

        return carry

    lax.fori_loop(0, (NSUPER + 2) // 2, body, 0)
    plsc.subcore_barrier()

    pltpu.sync_copy(acc_sh.at[pl.ds(s * 8, 8), :], obuf_v)
    pltpu.sync_copy(
        obuf_v, out_hbm.at[pl.ds(s * 8, 8), pl.ds(col0, COLS_PER_CORE)]
    )


def _tc_pool_kernel(x_ref, b_ref, out_ref):
    @pl.when(pl.program_id(0) == 0)
    def _():
        out_ref[...] = jnp.zeros_like(out_ref)

    seg = lax.broadcasted_iota(jnp.int32, (NUM_GRAPHS, TC_BS), 0)
    onehot = (b_ref[0, 0][None, :] == seg).astype(jnp.float32)
    out_ref[...] += jnp.dot(onehot, x_ref[...],
                            preferred_element_type=jnp.float32)


@jax.jit
def kernel(x, batch):
    batch = batch.astype(jnp.int32)
    batch3d_tc = lax.slice(batch, (0,), (TC_ROWS,)).reshape(TC_NB, 1, TC_BS)

    mesh = plsc.VectorSubcoreMesh(core_axis_name="c", subcore_axis_name="s")
    sc_out = pl.kernel(
        _sc_pool_kernel,
        out_type=jax.ShapeDtypeStruct((NUM_GRAPHS, D_FEAT), jnp.float32),
        mesh=mesh,
        scratch_types=[
            pltpu.VMEM((SC_CHUNKS_PER_SUB, 1, CHUNK), jnp.int32),
            pltpu.VMEM((2, SROWS, COLS_PER_CORE), jnp.float32),
            pltpu.VMEM((8, COLS_PER_CORE), jnp.float32),
            pltpu.VMEM_SHARED((NUM_GRAPHS, COLS_PER_CORE), jnp.float32),
            pltpu.SemaphoreType.DMA,
            pltpu.SemaphoreType.DMA,
            pltpu.SemaphoreType.DMA,
            pltpu.SemaphoreType.DMA,
            pltpu.SemaphoreType.DMA,
        ],
    )(x, batch)

    tc_out = pl.pallas_call(
        _tc_pool_kernel,
        grid=(TC_NB,),
        in_specs=[
            pl.BlockSpec((TC_BS, D_FEAT), lambda i: (i, 0)),
            pl.BlockSpec((1, 1, TC_BS), lambda i: (i, 0, 0)),
        ],
        out_specs=pl.BlockSpec((NUM_GRAPHS, D_FEAT), lambda i: (0, 0)),
        out_shape=jax.ShapeDtypeStruct((NUM_GRAPHS, D_FEAT), jnp.float32),
    )(x, batch3d_tc)

    return sc_out + tc_out


# speedup vs baseline: 1.0293x; 1.0293x over previous
import jax
import jax.numpy as jnp
from jax import lax
from jax.experimental import pallas as pl
from jax.experimental.pallas import tpu as pltpu, tpu_sc as plsc

NUM_NODES = 50000
D_FEAT = 256
NUM_GRAPHS = 128

NUM_CORES = 2
NUM_SUBCORES = 16
COLS_PER_CORE = D_FEAT // NUM_CORES

CHUNK = 80
NUM_CHUNKS = NUM_NODES // CHUNK
SUPER = 5
SROWS = SUPER * CHUNK
NSUPER = 3
SC_CHUNKS_PER_SUB = NSUPER * SUPER
SC_ROWS = NUM_SUBCORES * SC_CHUNKS_PER_SUB * CHUNK
SC_CHUNK0 = NUM_CHUNKS - NUM_SUBCORES * SC_CHUNKS_PER_SUB

TC_ROWS = NUM_NODES - SC_ROWS
TC_NB = 10
TC_BS = TC_ROWS // TC_NB


def _sc_pool_kernel(x_hbm, batch_hbm, out_hbm,
                    idx3d_v, rows_v, obuf_v, acc_sh,
                    gsem0, gsem1, ssem0, ssem1, isem):
    c = lax.axis_index("c")
    s = lax.axis_index("s")
    col0 = c * COLS_PER_CORE
    gsems = (gsem0, gsem1)
    ssems = (ssem0, ssem1)

    start = SC_CHUNK0 + s * SC_CHUNKS_PER_SUB

    def super_src(g):
        return x_hbm.at[pl.ds((start + g * SUPER) * CHUNK, SROWS),
                        pl.ds(col0, COLS_PER_CORE)]

    pltpu.async_copy(super_src(0), rows_v.at[0], gsems[0])

    for j in range(SC_CHUNKS_PER_SUB):
        pltpu.async_copy(
            batch_hbm.at[pl.ds((start + j) * CHUNK, CHUNK)],
            idx3d_v.at[j, 0], isem)

    zeros16 = jnp.zeros((16,), jnp.float32)
    for i in range(8):
        for j in range(COLS_PER_CORE // 16):
            obuf_v[i, pl.ds(j * 16, 16)] = zeros16
    pltpu.sync_copy(obuf_v, acc_sh.at[pl.ds(s * 8, 8), :])
    plsc.subcore_barrier()

    for j in range(SC_CHUNKS_PER_SUB):
        pltpu.make_async_copy(
            batch_hbm.at[pl.ds((start + j) * CHUNK, CHUNK)],
            idx3d_v.at[j, 0], isem).wait()

    def scatter_slices(g, b):
        for k in range(SUPER):
            yield (rows_v.at[b, pl.ds(k * CHUNK, CHUNK)],
                   acc_sh.at[idx3d_v.at[g * SUPER + k, 0]])

    def body(i, carry):
        for b in range(2):
            g = 2 * i + b
            nb = (b + 1) % 2

            @pl.when(g < NSUPER)
            def _():
                pltpu.make_async_copy(super_src(0), rows_v.at[b],
                                      gsems[b]).wait()
                for src, dst in scatter_slices(g, b):
                    pltpu.async_copy(src, dst, ssems[b], add=True)

            @pl.when((g >= 1) & (g - 1 < NSUPER))
            def _():
                for src, dst in scatter_slices(g - 1, nb):
                    pltpu.make_async_copy(src, dst, ssems[nb]).wait()

            @pl.when(g + 1 < NSUPER)
            def _():
                pltpu.async_copy(super_src(g + 1), rows_v.at[nb], gsems[nb])

        return carry

    lax.fori_loop(0, (NSUPER + 2) // 2, body, 0)
    plsc.subcore_barrier()

    pltpu.sync_copy(acc_sh.at[pl.ds(s * 8, 8), :], obuf_v)
    pltpu.sync_copy(
        obuf_v, out_hbm.at[pl.ds(s * 8, 8), pl.ds(col0, COLS_PER_CORE)]
    )


def _tc_pool_kernel(x_ref, b_ref, out_ref):
    @pl.when(pl.program_id(0) == 0)
    def _():
        out_ref[...] = jnp.zeros_like(out_ref)

    seg = lax.broadcasted_iota(jnp.int32, (NUM_GRAPHS, TC_BS), 0)
    onehot = (b_ref[0, 0][None, :] == seg).astype(jnp.float32)
    out_ref[...] += jnp.dot(onehot, x_ref[...],
                            preferred_element_type=jnp.float32)


@jax.jit
def kernel(x, batch):
    batch = batch.astype(jnp.int32)
    batch3d_tc = lax.slice(batch, (0,), (TC_ROWS,)).reshape(TC_NB, 1, TC_BS)

    mesh = plsc.VectorSubcoreMesh(core_axis_name="c", subcore_axis_name="s")
    sc_out = pl.kernel(
        _sc_pool_kernel,
        out_type=jax.ShapeDtypeStruct((NUM_GRAPHS, D_FEAT), jnp.float32),
        mesh=mesh,
        scratch_types=[
            pltpu.VMEM((SC_CHUNKS_PER_SUB, 1, CHUNK), jnp.int32),
            pltpu.VMEM((2, SROWS, COLS_PER_CORE), jnp.float32),
            pltpu.VMEM((8, COLS_PER_CORE), jnp.float32),
            pltpu.VMEM_SHARED((NUM_GRAPHS, COLS_PER_CORE), jnp.float32),
            pltpu.SemaphoreType.DMA,
            pltpu.SemaphoreType.DMA,
            pltpu.SemaphoreType.DMA,
            pltpu.SemaphoreType.DMA,
            pltpu.SemaphoreType.DMA,
        ],
    )(x, batch)

    tc_out = pl.pallas_call(
        _tc_pool_kernel,
        grid=(TC_NB,),
        in_specs=[
            pl.BlockSpec((TC_BS, D_FEAT), lambda i: (i, 0)),
            pl.BlockSpec((1, 1, TC_BS), lambda i: (i, 0, 0)),
        ],
        out_specs=pl.BlockSpec((NUM_GRAPHS, D_FEAT), lambda i: (0, 0)),
        out_shape=jax.ShapeDtypeStruct((NUM_GRAPHS, D_FEAT), jnp.float32),
    )(x, batch3d_tc)

    return sc_out + tc_out
